# baseline (speedup 1.0000x reference)
import functools

import jax
import jax.numpy as jnp
from jax import lax
from jax.experimental import pallas as pl
from jax.experimental.pallas import tpu as pltpu
from jax.experimental.pallas import tpu_sc as plsc

_NUM_CORES = 2
_NUM_SUBCORES = 16
_NUM_WORKERS = _NUM_CORES * _NUM_SUBCORES

_CHUNK = 1024


def _sc_gather(flat_idx, table):
    n = flat_idx.shape[0]
    _, d = table.shape
    n_per_w = n // _NUM_WORKERS
    n_chunks = n_per_w // _CHUNK
    mesh = plsc.VectorSubcoreMesh(core_axis_name="c", subcore_axis_name="s")

    @functools.partial(
        pl.kernel,
        out_type=jax.ShapeDtypeStruct((n, d), jnp.float32),
        mesh=mesh,
        scratch_types=[
            pltpu.VMEM((_CHUNK,), jnp.int32),
            pltpu.VMEM((_CHUNK, d), jnp.float32),
            pltpu.SemaphoreType.DMA,
        ],
        compiler_params=pltpu.CompilerParams(use_tc_tiling_on_sc=False),
    )
    def gather_kernel(idx_hbm, table_hbm, out_hbm, idx_v, rows_v, sem):
        wid = lax.axis_index("s") * _NUM_CORES + lax.axis_index("c")
        base = wid * n_per_w

        def body(i, _):
            off = base + i * _CHUNK
            pltpu.sync_copy(idx_hbm.at[pl.ds(off, _CHUNK)], idx_v)
            pltpu.async_copy(table_hbm.at[idx_v], rows_v, sem).wait()
            pltpu.sync_copy(rows_v, out_hbm.at[pl.ds(off, _CHUNK)])
            return 0

        lax.fori_loop(0, n_chunks, body, 0)

    return gather_kernel(flat_idx, table)


def _tc_project(emb, w):
    n, d = emb.shape
    dm = w.shape[1]
    blk = 2048

    def mm(emb_ref, w_ref, out_ref):
        out_ref[...] = jnp.dot(
            emb_ref[...], w_ref[...], preferred_element_type=jnp.float32
        )

    return pl.pallas_call(
        mm,
        grid=(n // blk,),
        in_specs=[
            pl.BlockSpec((blk, d), lambda i: (i, 0)),
            pl.BlockSpec((d, dm), lambda i: (0, 0)),
        ],
        out_specs=pl.BlockSpec((blk, dm), lambda i: (i, 0)),
        out_shape=jax.ShapeDtypeStruct((n, dm), jnp.float32),
    )(emb, w)


def kernel(x, table, W):
    b, l = x.shape
    dm = W.shape[1]
    flat_idx = x.reshape(b * l).astype(jnp.int32)
    emb = _sc_gather(flat_idx, table)
    out = _tc_project(emb, W)
    return out.reshape(b, l, dm)


# speedup vs baseline: 1.1284x; 1.1284x over previous
import functools

import jax
import jax.numpy as jnp
from jax import lax
from jax.experimental import pallas as pl
from jax.experimental.pallas import tpu as pltpu
from jax.experimental.pallas import tpu_sc as plsc

_NUM_CORES = 2
_NUM_SUBCORES = 16
_NUM_WORKERS = _NUM_CORES * _NUM_SUBCORES

_CHUNK = 1024


def _sc_gather(flat_idx, table):
    n = flat_idx.shape[0]
    _, d = table.shape
    n_per_w = n // _NUM_WORKERS
    n_chunks = n_per_w // _CHUNK
    mesh = plsc.VectorSubcoreMesh(core_axis_name="c", subcore_axis_name="s")

    @functools.partial(
        pl.kernel,
        out_type=jax.ShapeDtypeStruct((n, d), jnp.float32),
        mesh=mesh,
        scratch_types=[
            pltpu.VMEM((_CHUNK,), jnp.int32),
            pltpu.VMEM((_CHUNK, d), jnp.float32),
            pltpu.SemaphoreType.DMA,
        ],
        compiler_params=pltpu.CompilerParams(use_tc_tiling_on_sc=False),
    )
    def gather_kernel(idx_hbm, table_hbm, out_hbm, idx_v, rows_v, sem):
        wid = lax.axis_index("s") * _NUM_CORES + lax.axis_index("c")
        base = wid * n_per_w

        def body(i, _):
            off = base + i * _CHUNK
            pltpu.sync_copy(idx_hbm.at[pl.ds(off, _CHUNK)], idx_v)
            pltpu.async_copy(table_hbm.at[idx_v], rows_v, sem).wait()
            pltpu.sync_copy(rows_v, out_hbm.at[pl.ds(off, _CHUNK)])
            return 0

        lax.fori_loop(0, n_chunks, body, 0)

    return gather_kernel(flat_idx, table)


def _tc_project(emb, w):
    n, d = emb.shape
    dm = w.shape[1]
    blk = 8192

    def mm(emb_ref, w_ref, out_ref):
        out_ref[...] = jnp.dot(
            emb_ref[...], w_ref[...], preferred_element_type=jnp.float32
        )

    return pl.pallas_call(
        mm,
        grid=(n // blk,),
        in_specs=[
            pl.BlockSpec((blk, d), lambda i: (i, 0)),
            pl.BlockSpec((d, dm), lambda i: (0, 0)),
        ],
        out_specs=pl.BlockSpec((blk, dm), lambda i: (i, 0)),
        out_shape=jax.ShapeDtypeStruct((n, dm), jnp.float32),
    )(emb, w)


def kernel(x, table, W):
    b, l = x.shape
    dm = W.shape[1]
    flat_idx = x.reshape(b * l).astype(jnp.int32)
    emb = _sc_gather(flat_idx, table)
    out = _tc_project(emb, W)
    return out.reshape(b, l, dm)


# speedup vs baseline: 1.1292x; 1.0007x over previous
import functools

import jax
import jax.numpy as jnp
from jax import lax
from jax.experimental import pallas as pl
from jax.experimental.pallas import tpu as pltpu
from jax.experimental.pallas import tpu_sc as plsc

_NUM_CORES = 2
_NUM_SUBCORES = 16
_NUM_WORKERS = _NUM_CORES * _NUM_SUBCORES

_CHUNK = 1024


def _sc_gather(flat_idx, table):
    n = flat_idx.shape[0]
    _, d = table.shape
    n_per_w = n // _NUM_WORKERS
    n_chunks = n_per_w // _CHUNK
    mesh = plsc.VectorSubcoreMesh(core_axis_name="c", subcore_axis_name="s")

    @functools.partial(
        pl.kernel,
        out_type=jax.ShapeDtypeStruct((n, d), jnp.float32),
        mesh=mesh,
        scratch_types=[
            pltpu.VMEM((_CHUNK,), jnp.int32),
            pltpu.VMEM((_CHUNK, d), jnp.float32),
            pltpu.SemaphoreType.DMA,
        ],
        compiler_params=pltpu.CompilerParams(use_tc_tiling_on_sc=False),
    )
    def gather_kernel(idx_hbm, table_hbm, out_hbm, idx_v, rows_v, sem):
        wid = lax.axis_index("s") * _NUM_CORES + lax.axis_index("c")
        base = wid * n_per_w

        def body(i, _):
            off = base + i * _CHUNK
            pltpu.sync_copy(idx_hbm.at[pl.ds(off, _CHUNK)], idx_v)
            pltpu.async_copy(table_hbm.at[idx_v], rows_v, sem).wait()
            pltpu.sync_copy(rows_v, out_hbm.at[pl.ds(off, _CHUNK)])
            return 0

        lax.fori_loop(0, n_chunks, body, 0)

    return gather_kernel(flat_idx, table)


def _tc_project(emb, w, b, l):
    n, d = emb.shape
    dm = w.shape[1]
    bb = 64

    def mm(emb_ref, w_ref, out_ref):
        acc = jnp.dot(emb_ref[...], w_ref[...], preferred_element_type=jnp.float32)
        out_ref[...] = acc.reshape(bb, l, dm)

    return pl.pallas_call(
        mm,
        grid=(b // bb,),
        in_specs=[
            pl.BlockSpec((bb * l, d), lambda i: (i, 0)),
            pl.BlockSpec((d, dm), lambda i: (0, 0)),
        ],
        out_specs=pl.BlockSpec((bb, l, dm), lambda i: (i, 0, 0)),
        out_shape=jax.ShapeDtypeStruct((b, l, dm), jnp.float32),
    )(emb, w)


def kernel(x, table, W):
    b, l = x.shape
    flat_idx = x.reshape(b * l).astype(jnp.int32)
    emb = _sc_gather(flat_idx, table)
    return _tc_project(emb, W, b, l)


# speedup vs baseline: 2.3286x; 2.0622x over previous
import functools

import jax
import jax.numpy as jnp
from jax import lax
from jax.experimental import pallas as pl
from jax.experimental.pallas import tpu as pltpu
from jax.experimental.pallas import tpu_sc as plsc

_NUM_CORES = 2
_NUM_SUBCORES = 16
_NUM_WORKERS = _NUM_CORES * _NUM_SUBCORES

_CHUNK = 512


def _tc_project_table(table_t, w):
    d, v = table_t.shape
    dm = w.shape[1]
    cb = 6400

    def mm(t_ref, w_ref, out_ref):
        out_ref[...] = jax.lax.dot_general(
            t_ref[...],
            w_ref[...],
            dimension_numbers=(((0,), (0,)), ((), ())),
            preferred_element_type=jnp.float32,
        )

    return pl.pallas_call(
        mm,
        grid=((v + cb - 1) // cb,),
        in_specs=[
            pl.BlockSpec((d, cb), lambda i: (0, i)),
            pl.BlockSpec((d, dm), lambda i: (0, 0)),
        ],
        out_specs=pl.BlockSpec((cb, dm), lambda i: (i, 0)),
        out_shape=jax.ShapeDtypeStruct((v, dm), jnp.float32),
    )(table_t, w)


def _sc_gather(flat_idx, p):
    n = flat_idx.shape[0]
    dm = p.shape[1]
    n_per_w = n // _NUM_WORKERS
    n_chunks = n_per_w // _CHUNK
    mesh = plsc.VectorSubcoreMesh(core_axis_name="c", subcore_axis_name="s")

    @functools.partial(
        pl.kernel,
        out_type=jax.ShapeDtypeStruct((n, dm), jnp.float32),
        mesh=mesh,
        scratch_types=[
            pltpu.VMEM((_CHUNK,), jnp.int32),
            pltpu.VMEM((_CHUNK, dm), jnp.float32),
            pltpu.SemaphoreType.DMA,
        ],
        compiler_params=pltpu.CompilerParams(use_tc_tiling_on_sc=True),
    )
    def gather_kernel(idx_hbm, p_hbm, out_hbm, idx_v, rows_v, sem):
        wid = lax.axis_index("s") * _NUM_CORES + lax.axis_index("c")
        base = wid * n_per_w

        def body(i, _):
            off = base + i * _CHUNK
            pltpu.sync_copy(idx_hbm.at[pl.ds(off, _CHUNK)], idx_v)
            pltpu.async_copy(p_hbm.at[idx_v], rows_v, sem).wait()
            pltpu.sync_copy(rows_v, out_hbm.at[pl.ds(off, _CHUNK)])
            return 0

        lax.fori_loop(0, n_chunks, body, 0)

    return gather_kernel(flat_idx, p)


def kernel(x, table, W):
    b, l = x.shape
    dm = W.shape[1]
    flat_idx = x.reshape(b * l).astype(jnp.int32)
    p = _tc_project_table(table.T, W)
    out = _sc_gather(flat_idx, p)
    return out.reshape(b, l, dm)


# speedup vs baseline: 2.4722x; 1.0616x over previous
import functools

import jax
import jax.numpy as jnp
from jax import lax
from jax.experimental import pallas as pl
from jax.experimental.pallas import tpu as pltpu
from jax.experimental.pallas import tpu_sc as plsc

_NUM_CORES = 2
_NUM_SUBCORES = 16
_NUM_WORKERS = _NUM_CORES * _NUM_SUBCORES

_CHUNK = 400


def _tc_project_table(table_t, w):
    d, v = table_t.shape
    dm = w.shape[1]
    cb = 6400

    def mm(t_ref, w_ref, out_ref):
        out_ref[...] = jax.lax.dot_general(
            t_ref[...],
            w_ref[...],
            dimension_numbers=(((0,), (0,)), ((), ())),
            preferred_element_type=jnp.float32,
        )

    return pl.pallas_call(
        mm,
        grid=((v + cb - 1) // cb,),
        in_specs=[
            pl.BlockSpec((d, cb), lambda i: (0, i)),
            pl.BlockSpec((d, dm), lambda i: (0, 0)),
        ],
        out_specs=pl.BlockSpec((cb, dm), lambda i: (i, 0)),
        out_shape=jax.ShapeDtypeStruct((v, dm), jnp.float32),
    )(table_t, w)


def _sc_gather(flat_idx, p):
    n = flat_idx.shape[0]
    dm = p.shape[1]
    n_per_w = n // _NUM_WORKERS
    n_chunks = n_per_w // _CHUNK
    mesh = plsc.VectorSubcoreMesh(core_axis_name="c", subcore_axis_name="s")

    @functools.partial(
        pl.kernel,
        out_type=jax.ShapeDtypeStruct((n, dm), jnp.float32),
        mesh=mesh,
        scratch_types=[
            pltpu.VMEM((n_per_w,), jnp.int32),
            pltpu.VMEM((_CHUNK, dm), jnp.float32),
            pltpu.VMEM((_CHUNK, dm), jnp.float32),
            pltpu.SemaphoreType.DMA,
            pltpu.SemaphoreType.DMA,
        ],
        compiler_params=pltpu.CompilerParams(use_tc_tiling_on_sc=True),
    )
    def gather_kernel(idx_hbm, p_hbm, out_hbm, idx_all, rows0, rows1, sem0, sem1):
        wid = lax.axis_index("s") * _NUM_CORES + lax.axis_index("c")
        base = wid * n_per_w

        pltpu.sync_copy(idx_hbm.at[pl.ds(base, n_per_w)], idx_all)

        def fire(c, rows, sem):
            pltpu.async_copy(p_hbm.at[idx_all.at[pl.ds(c * _CHUNK, _CHUNK)]], rows, sem)

        def drain(rows, sem):
            pltpu.make_async_copy(p_hbm.at[pl.ds(0, _CHUNK)], rows, sem).wait()

        def store(c, rows):
            pltpu.sync_copy(rows, out_hbm.at[pl.ds(base + c * _CHUNK, _CHUNK)])

        fire(0, rows0, sem0)

        def body(j, _):
            c0 = 2 * j
            fire(c0 + 1, rows1, sem1)
            drain(rows0, sem0)
            store(c0, rows0)
            fire(c0 + 2, rows0, sem0)
            drain(rows1, sem1)
            store(c0 + 1, rows1)
            return 0

        lax.fori_loop(0, n_chunks // 2 - 1, body, 0)

        c_last = n_chunks - 2
        fire(c_last + 1, rows1, sem1)
        drain(rows0, sem0)
        store(c_last, rows0)
        drain(rows1, sem1)
        store(c_last + 1, rows1)

    return gather_kernel(flat_idx, p)


def kernel(x, table, W):
    b, l = x.shape
    dm = W.shape[1]
    flat_idx = x.reshape(b * l).astype(jnp.int32)
    p = _tc_project_table(table.T, W)
    out = _sc_gather(flat_idx, p)
    return out.reshape(b, l, dm)


# speedup vs baseline: 2.6604x; 1.0761x over previous
import functools

import jax
import jax.numpy as jnp
from jax import lax
from jax.experimental import pallas as pl
from jax.experimental.pallas import tpu as pltpu
from jax.experimental.pallas import tpu_sc as plsc

_NUM_CORES = 2
_NUM_SUBCORES = 16
_NUM_WORKERS = _NUM_CORES * _NUM_SUBCORES

_CHUNK = 400


def _tc_project_table(table_t, w):
    d, v = table_t.shape
    dm = w.shape[1]
    cb = 12800

    def mm(t_ref, w_ref, out_ref):
        out_ref[...] = jax.lax.dot_general(
            t_ref[...],
            w_ref[...],
            dimension_numbers=(((0,), (0,)), ((), ())),
            preferred_element_type=jnp.float32,
        )

    return pl.pallas_call(
        mm,
        grid=((v + cb - 1) // cb,),
        in_specs=[
            pl.BlockSpec((d, cb), lambda i: (0, i)),
            pl.BlockSpec((d, dm), lambda i: (0, 0)),
        ],
        out_specs=pl.BlockSpec((cb, dm), lambda i: (i, 0)),
        out_shape=jax.ShapeDtypeStruct((v, dm), jnp.float32),
    )(table_t, w)


def _sc_gather(flat_idx, p):
    n = flat_idx.shape[0]
    dm = p.shape[1]
    n_per_w = n // _NUM_WORKERS
    n_chunks = n_per_w // _CHUNK
    mesh = plsc.VectorSubcoreMesh(core_axis_name="c", subcore_axis_name="s")

    @functools.partial(
        pl.kernel,
        out_type=jax.ShapeDtypeStruct((n, dm), jnp.float32),
        mesh=mesh,
        scratch_types=[
            pltpu.VMEM((n_per_w,), jnp.int32),
            pltpu.VMEM((_CHUNK, dm), jnp.float32),
            pltpu.VMEM((_CHUNK, dm), jnp.float32),
            pltpu.SemaphoreType.DMA,
            pltpu.SemaphoreType.DMA,
        ],
        compiler_params=pltpu.CompilerParams(use_tc_tiling_on_sc=True),
    )
    def gather_kernel(idx_hbm, p_hbm, out_hbm, idx_all, rows0, rows1, sem0, sem1):
        wid = lax.axis_index("s") * _NUM_CORES + lax.axis_index("c")
        base = wid * n_per_w

        pltpu.sync_copy(idx_hbm.at[pl.ds(base, n_per_w)], idx_all)

        def fire(c, rows, sem):
            pltpu.async_copy(p_hbm.at[idx_all.at[pl.ds(c * _CHUNK, _CHUNK)]], rows, sem)

        def drain(rows, sem):
            pltpu.make_async_copy(p_hbm.at[pl.ds(0, _CHUNK)], rows, sem).wait()

        def store(c, rows):
            pltpu.sync_copy(rows, out_hbm.at[pl.ds(base + c * _CHUNK, _CHUNK)])

        fire(0, rows0, sem0)

        def body(j, _):
            c0 = 2 * j
            fire(c0 + 1, rows1, sem1)
            drain(rows0, sem0)
            store(c0, rows0)
            fire(c0 + 2, rows0, sem0)
            drain(rows1, sem1)
            store(c0 + 1, rows1)
            return 0

        lax.fori_loop(0, n_chunks // 2 - 1, body, 0)

        c_last = n_chunks - 2
        fire(c_last + 1, rows1, sem1)
        drain(rows0, sem0)
        store(c_last, rows0)
        drain(rows1, sem1)
        store(c_last + 1, rows1)

    return gather_kernel(flat_idx, p)


def kernel(x, table, W):
    b, l = x.shape
    dm = W.shape[1]
    flat_idx = x.reshape(b * l).astype(jnp.int32)
    p = _tc_project_table(table.T, W)
    out = _sc_gather(flat_idx, p)
    return out.reshape(b, l, dm)


# speedup vs baseline: 2.7014x; 1.0154x over previous
import functools

import jax
import jax.numpy as jnp
from jax import lax
from jax.experimental import pallas as pl
from jax.experimental.pallas import tpu as pltpu
from jax.experimental.pallas import tpu_sc as plsc

_NUM_CORES = 2
_NUM_SUBCORES = 16
_NUM_WORKERS = _NUM_CORES * _NUM_SUBCORES

_CHUNK = 400


def _tc_project_table(table_t, w):
    d, v = table_t.shape
    dm = w.shape[1]
    cb = 25600

    def mm(t_ref, w_ref, out_ref):
        out_ref[...] = jax.lax.dot_general(
            t_ref[...],
            w_ref[...],
            dimension_numbers=(((0,), (0,)), ((), ())),
            preferred_element_type=jnp.float32,
        )

    return pl.pallas_call(
        mm,
        grid=((v + cb - 1) // cb,),
        in_specs=[
            pl.BlockSpec((d, cb), lambda i: (0, i)),
            pl.BlockSpec((d, dm), lambda i: (0, 0)),
        ],
        out_specs=pl.BlockSpec((cb, dm), lambda i: (i, 0)),
        out_shape=jax.ShapeDtypeStruct((v, dm), jnp.float32),
    )(table_t, w)


def _sc_gather(flat_idx, p):
    n = flat_idx.shape[0]
    dm = p.shape[1]
    n_per_w = n // _NUM_WORKERS
    n_chunks = n_per_w // _CHUNK
    mesh = plsc.VectorSubcoreMesh(core_axis_name="c", subcore_axis_name="s")

    @functools.partial(
        pl.kernel,
        out_type=jax.ShapeDtypeStruct((n, dm), jnp.float32),
        mesh=mesh,
        scratch_types=[
            pltpu.VMEM((n_per_w,), jnp.int32),
            pltpu.VMEM((_CHUNK, dm), jnp.float32),
            pltpu.VMEM((_CHUNK, dm), jnp.float32),
            pltpu.SemaphoreType.DMA,
            pltpu.SemaphoreType.DMA,
        ],
        compiler_params=pltpu.CompilerParams(use_tc_tiling_on_sc=True),
    )
    def gather_kernel(idx_hbm, p_hbm, out_hbm, idx_all, rows0, rows1, sem0, sem1):
        wid = lax.axis_index("s") * _NUM_CORES + lax.axis_index("c")
        base = wid * n_per_w

        pltpu.sync_copy(idx_hbm.at[pl.ds(base, n_per_w)], idx_all)

        def fire(c, rows, sem):
            pltpu.async_copy(p_hbm.at[idx_all.at[pl.ds(c * _CHUNK, _CHUNK)]], rows, sem)

        def drain(rows, sem):
            pltpu.make_async_copy(p_hbm.at[pl.ds(0, _CHUNK)], rows, sem).wait()

        def store(c, rows):
            pltpu.sync_copy(rows, out_hbm.at[pl.ds(base + c * _CHUNK, _CHUNK)])

        fire(0, rows0, sem0)

        def body(j, _):
            c0 = 2 * j
            fire(c0 + 1, rows1, sem1)
            drain(rows0, sem0)
            store(c0, rows0)
            fire(c0 + 2, rows0, sem0)
            drain(rows1, sem1)
            store(c0 + 1, rows1)
            return 0

        lax.fori_loop(0, n_chunks // 2 - 1, body, 0)

        c_last = n_chunks - 2
        fire(c_last + 1, rows1, sem1)
        drain(rows0, sem0)
        store(c_last, rows0)
        drain(rows1, sem1)
        store(c_last + 1, rows1)

    return gather_kernel(flat_idx, p)


def kernel(x, table, W):
    b, l = x.shape
    dm = W.shape[1]
    flat_idx = x.reshape(b * l).astype(jnp.int32)
    p = _tc_project_table(table.T, W)
    out = _sc_gather(flat_idx, p)
    return out.reshape(b, l, dm)
